# baseline (speedup 1.0000x reference)
import functools

import jax
import jax.numpy as jnp
from jax import lax
from jax.experimental import pallas as pl
from jax.experimental.pallas import tpu as pltpu
from jax.experimental.pallas import tpu_sc as plsc

N_NODES = 10000
N_EDGES = 320000
FEAT = 128
FHALF = FEAT // 2

NC = 2
NS = 16
CHUNK = 64
TOTAL_CH = N_EDGES // CHUNK
CH_PER_TILE = TOTAL_CH // NS
TAIL_CH = TOTAL_CH - NS * CH_PER_TILE
NBUF = 12
NROUND = CH_PER_TILE // NBUF

ROWS_PER_TILE = 640
ROWS_LAST = N_NODES - 15 * ROWS_PER_TILE

_mesh = plsc.VectorSubcoreMesh(core_axis_name="c", subcore_axis_name="s")


@functools.partial(
    pl.kernel,
    mesh=_mesh,
    out_type=jax.ShapeDtypeStruct((N_NODES, FEAT), jnp.float32),
    scratch_types=[
        pltpu.VMEM((NBUF, CHUNK), jnp.int32),
        pltpu.VMEM((NBUF, CHUNK, FHALF), jnp.float32),
        pltpu.VMEM((CHUNK, FHALF), jnp.float32),
        pltpu.VMEM_SHARED((N_NODES, FHALF), jnp.float32),
        pltpu.SemaphoreType.DMA((NBUF,)),
        pltpu.SemaphoreType.DMA((NBUF,)),
        pltpu.SemaphoreType.DMA,
    ],
    compiler_params=pltpu.CompilerParams(use_tc_tiling_on_sc=False),
)
def _sc_scatter(dst_hbm, w_hbm, out_hbm, idx_v, rows_v, zbuf, acc_sh,
                fsem, ssem, zsem):
    cid = lax.axis_index("c")
    sid = lax.axis_index("s")
    fbase = cid * FHALF
    rbase = sid * ROWS_PER_TILE
    cbase = sid * CH_PER_TILE

    def _fetch_idx(g, b):
        return pltpu.make_async_copy(
            dst_hbm.at[0, pl.ds(g * CHUNK, CHUNK)], idx_v.at[b], fsem.at[b]
        )

    def _fetch_rows(g, b):
        return pltpu.make_async_copy(
            w_hbm.at[pl.ds(g * CHUNK, CHUNK), pl.ds(fbase, FHALF)],
            rows_v.at[b],
            fsem.at[b],
        )

    def _scatter(b):
        return pltpu.make_async_copy(
            rows_v.at[b], acc_sh.at[idx_v.at[b]], ssem.at[b]
        )

    def _start_fetch(g, b):
        _fetch_idx(g, b).start()
        _fetch_rows(g, b).start()

    def _wait_fetch(g, b):
        _fetch_idx(g, b).wait()
        _fetch_rows(g, b).wait()

    for b in range(NBUF):
        _start_fetch(cbase + b, b)

    zvec = jnp.zeros((16,), jnp.float32)

    def _zrow(r, carry):
        for c in range(FHALF // 16):
            zbuf[r, pl.ds(c * 16, 16)] = zvec
        return carry

    lax.fori_loop(0, CHUNK, _zrow, 0)

    def _zcopy(k):
        return pltpu.make_async_copy(
            zbuf, acc_sh.at[pl.ds(rbase + k * CHUNK, CHUNK), :], zsem
        )

    _zn = ROWS_PER_TILE // CHUNK
    _zn_last = ROWS_LAST // CHUNK

    @pl.when(sid < NS - 1)
    def _zero_main():
        for k in range(_zn):
            _zcopy(k).start()
        for k in range(_zn):
            _zcopy(k).wait()

    @pl.when(sid == NS - 1)
    def _zero_last():
        for k in range(_zn_last):
            _zcopy(k).start()
        pltpu.make_async_copy(
            zbuf.at[pl.ds(0, ROWS_LAST % CHUNK)],
            acc_sh.at[pl.ds(rbase + _zn_last * CHUNK, ROWS_LAST % CHUNK), :],
            zsem,
        ).start()
        for k in range(_zn_last):
            _zcopy(k).wait()
        pltpu.make_async_copy(
            zbuf.at[pl.ds(0, ROWS_LAST % CHUNK)],
            acc_sh.at[pl.ds(rbase + _zn_last * CHUNK, ROWS_LAST % CHUNK), :],
            zsem,
        ).wait()

    plsc.subcore_barrier()

    def round_body(t, carry):
        t0 = t * NBUF
        for b in range(NBUF):
            j = t0 + b
            _wait_fetch(cbase + j, b)
            _scatter(b).start(add=True)
        for b in range(NBUF):
            j = t0 + b

            @pl.when(j + NBUF < CH_PER_TILE)
            def _refill(j=j, b=b):
                _scatter(b).wait()
                _start_fetch(cbase + j + NBUF, b)

        return carry

    lax.fori_loop(0, NROUND, round_body, 0)

    for b in range(NBUF):
        _scatter(b).wait()

    @pl.when(sid < TAIL_CH)
    def _tail():
        g = NS * CH_PER_TILE + sid
        pltpu.sync_copy(dst_hbm.at[0, pl.ds(g * CHUNK, CHUNK)], idx_v.at[0])
        pltpu.sync_copy(
            w_hbm.at[pl.ds(g * CHUNK, CHUNK), pl.ds(fbase, FHALF)],
            rows_v.at[0],
        )
        pltpu.sync_copy(rows_v.at[0], acc_sh.at[idx_v.at[0]], add=True)

    plsc.subcore_barrier()

    @pl.when(sid < NS - 1)
    def _flush_main():
        pltpu.sync_copy(
            acc_sh.at[pl.ds(rbase, ROWS_PER_TILE), :],
            out_hbm.at[pl.ds(rbase, ROWS_PER_TILE), pl.ds(fbase, FHALF)],
        )

    @pl.when(sid == NS - 1)
    def _flush_last():
        pltpu.sync_copy(
            acc_sh.at[pl.ds(15 * ROWS_PER_TILE, ROWS_LAST), :],
            out_hbm.at[pl.ds(15 * ROWS_PER_TILE, ROWS_LAST),
                       pl.ds(fbase, FHALF)],
        )


def kernel(edge, edge_w, N, E, out_features):
    return _sc_scatter(edge.astype(jnp.int32), edge_w)


# speedup vs baseline: 1.0031x; 1.0031x over previous
import functools

import jax
import jax.numpy as jnp
from jax import lax
from jax.experimental import pallas as pl
from jax.experimental.pallas import tpu as pltpu
from jax.experimental.pallas import tpu_sc as plsc

N_NODES = 10000
N_EDGES = 320000
FEAT = 128
FHALF = FEAT // 2

NC = 2
NS = 16
CHUNK = 128
TOTAL_CH = N_EDGES // CHUNK
CH_PER_TILE = TOTAL_CH // NS
TAIL_CH = TOTAL_CH - NS * CH_PER_TILE
NBUF = 6
NROUND = CH_PER_TILE // NBUF

ROWS_PER_TILE = 640
ROWS_LAST = N_NODES - 15 * ROWS_PER_TILE

_mesh = plsc.VectorSubcoreMesh(core_axis_name="c", subcore_axis_name="s")


@functools.partial(
    pl.kernel,
    mesh=_mesh,
    out_type=jax.ShapeDtypeStruct((N_NODES, FEAT), jnp.float32),
    scratch_types=[
        pltpu.VMEM((NBUF, CHUNK), jnp.int32),
        pltpu.VMEM((NBUF, CHUNK, FHALF), jnp.float32),
        pltpu.VMEM((CHUNK, FHALF), jnp.float32),
        pltpu.VMEM_SHARED((N_NODES, FHALF), jnp.float32),
        pltpu.SemaphoreType.DMA((NBUF,)),
        pltpu.SemaphoreType.DMA((NBUF,)),
        pltpu.SemaphoreType.DMA,
    ],
    compiler_params=pltpu.CompilerParams(use_tc_tiling_on_sc=False),
)
def _sc_scatter(dst_hbm, w_hbm, out_hbm, idx_v, rows_v, zbuf, acc_sh,
                fsem, ssem, zsem):
    cid = lax.axis_index("c")
    sid = lax.axis_index("s")
    fbase = cid * FHALF
    rbase = sid * ROWS_PER_TILE
    cbase = sid * CH_PER_TILE

    def _fetch_idx(g, b):
        return pltpu.make_async_copy(
            dst_hbm.at[0, pl.ds(g * CHUNK, CHUNK)], idx_v.at[b], fsem.at[b]
        )

    def _fetch_rows(g, b):
        return pltpu.make_async_copy(
            w_hbm.at[pl.ds(g * CHUNK, CHUNK), pl.ds(fbase, FHALF)],
            rows_v.at[b],
            fsem.at[b],
        )

    def _scatter(b):
        return pltpu.make_async_copy(
            rows_v.at[b], acc_sh.at[idx_v.at[b]], ssem.at[b]
        )

    def _start_fetch(g, b):
        _fetch_idx(g, b).start()
        _fetch_rows(g, b).start()

    def _wait_fetch(g, b):
        _fetch_idx(g, b).wait()
        _fetch_rows(g, b).wait()

    for b in range(NBUF):
        _start_fetch(cbase + b, b)

    zvec = jnp.zeros((16,), jnp.float32)

    def _zrow(r, carry):
        for c in range(FHALF // 16):
            zbuf[r, pl.ds(c * 16, 16)] = zvec
        return carry

    lax.fori_loop(0, CHUNK, _zrow, 0)

    def _zcopy(k):
        return pltpu.make_async_copy(
            zbuf, acc_sh.at[pl.ds(rbase + k * CHUNK, CHUNK), :], zsem
        )

    _zn = ROWS_PER_TILE // CHUNK
    _zn_last = ROWS_LAST // CHUNK

    @pl.when(sid < NS - 1)
    def _zero_main():
        for k in range(_zn):
            _zcopy(k).start()
        for k in range(_zn):
            _zcopy(k).wait()

    @pl.when(sid == NS - 1)
    def _zero_last():
        for k in range(_zn_last):
            _zcopy(k).start()
        pltpu.make_async_copy(
            zbuf.at[pl.ds(0, ROWS_LAST % CHUNK)],
            acc_sh.at[pl.ds(rbase + _zn_last * CHUNK, ROWS_LAST % CHUNK), :],
            zsem,
        ).start()
        for k in range(_zn_last):
            _zcopy(k).wait()
        pltpu.make_async_copy(
            zbuf.at[pl.ds(0, ROWS_LAST % CHUNK)],
            acc_sh.at[pl.ds(rbase + _zn_last * CHUNK, ROWS_LAST % CHUNK), :],
            zsem,
        ).wait()

    plsc.subcore_barrier()

    def round_body(t, carry):
        t0 = t * NBUF
        for b in range(NBUF):
            j = t0 + b
            _wait_fetch(cbase + j, b)
            _scatter(b).start(add=True)
        for b in range(NBUF):
            j = t0 + b

            @pl.when(j + NBUF < CH_PER_TILE)
            def _refill(j=j, b=b):
                _scatter(b).wait()
                _start_fetch(cbase + j + NBUF, b)

        return carry

    lax.fori_loop(0, NROUND, round_body, 0)

    for b in range(NBUF):
        _scatter(b).wait()

    @pl.when(sid < TAIL_CH)
    def _tail():
        g = NS * CH_PER_TILE + sid
        pltpu.sync_copy(dst_hbm.at[0, pl.ds(g * CHUNK, CHUNK)], idx_v.at[0])
        pltpu.sync_copy(
            w_hbm.at[pl.ds(g * CHUNK, CHUNK), pl.ds(fbase, FHALF)],
            rows_v.at[0],
        )
        pltpu.sync_copy(rows_v.at[0], acc_sh.at[idx_v.at[0]], add=True)

    plsc.subcore_barrier()

    @pl.when(sid < NS - 1)
    def _flush_main():
        pltpu.sync_copy(
            acc_sh.at[pl.ds(rbase, ROWS_PER_TILE), :],
            out_hbm.at[pl.ds(rbase, ROWS_PER_TILE), pl.ds(fbase, FHALF)],
        )

    @pl.when(sid == NS - 1)
    def _flush_last():
        pltpu.sync_copy(
            acc_sh.at[pl.ds(15 * ROWS_PER_TILE, ROWS_LAST), :],
            out_hbm.at[pl.ds(15 * ROWS_PER_TILE, ROWS_LAST),
                       pl.ds(fbase, FHALF)],
        )


def kernel(edge, edge_w, N, E, out_features):
    if edge.dtype != jnp.int32:
        edge = edge.astype(jnp.int32)
    return _sc_scatter(edge, edge_w)


# speedup vs baseline: 1.0102x; 1.0071x over previous
import functools

import jax
import jax.numpy as jnp
from jax import lax
from jax.experimental import pallas as pl
from jax.experimental.pallas import tpu as pltpu
from jax.experimental.pallas import tpu_sc as plsc

N_NODES = 10000
N_EDGES = 320000
FEAT = 128
FHALF = FEAT // 2

NC = 2
NS = 16
CHUNK = 128
TOTAL_CH = N_EDGES // CHUNK
CH_PER_TILE = TOTAL_CH // NS
TAIL_CH = TOTAL_CH - NS * CH_PER_TILE
NBUF = 6
NROUND = CH_PER_TILE // NBUF

ROWS_PER_TILE = 640
ROWS_LAST = N_NODES - 15 * ROWS_PER_TILE

_mesh = plsc.VectorSubcoreMesh(core_axis_name="c", subcore_axis_name="s")


@functools.partial(
    pl.kernel,
    mesh=_mesh,
    out_type=jax.ShapeDtypeStruct((N_NODES, FEAT), jnp.float32),
    scratch_types=[
        pltpu.VMEM((NBUF, CHUNK), jnp.int32),
        pltpu.VMEM((NBUF, CHUNK, FHALF), jnp.float32),
        pltpu.VMEM((CHUNK, FHALF), jnp.float32),
        pltpu.VMEM((1, CHUNK), jnp.int32),
        pltpu.VMEM((CHUNK, FHALF), jnp.float32),
        pltpu.VMEM_SHARED((N_NODES, FHALF), jnp.float32),
        pltpu.SemaphoreType.DMA((NBUF,)),
        pltpu.SemaphoreType.DMA((NBUF,)),
        pltpu.SemaphoreType.DMA,
        pltpu.SemaphoreType.DMA,
    ],
    compiler_params=pltpu.CompilerParams(use_tc_tiling_on_sc=False),
)
def _sc_scatter(dst_hbm, w_hbm, out_hbm, idx_v, rows_v, zbuf, tidx_v, trows_v,
                acc_sh, fsem, ssem, zsem, tsem):
    cid = lax.axis_index("c")
    sid = lax.axis_index("s")
    fbase = cid * FHALF
    rbase = sid * ROWS_PER_TILE
    cbase = sid * CH_PER_TILE

    def _fetch_idx(g, b):
        return pltpu.make_async_copy(
            dst_hbm.at[0, pl.ds(g * CHUNK, CHUNK)], idx_v.at[b], fsem.at[b]
        )

    def _fetch_rows(g, b):
        return pltpu.make_async_copy(
            w_hbm.at[pl.ds(g * CHUNK, CHUNK), pl.ds(fbase, FHALF)],
            rows_v.at[b],
            fsem.at[b],
        )

    def _scatter(b):
        return pltpu.make_async_copy(
            rows_v.at[b], acc_sh.at[idx_v.at[b]], ssem.at[b]
        )

    def _start_fetch(g, b):
        _fetch_idx(g, b).start()
        _fetch_rows(g, b).start()

    def _wait_fetch(g, b):
        _fetch_idx(g, b).wait()
        _fetch_rows(g, b).wait()

    for b in range(NBUF):
        _start_fetch(cbase + b, b)

    gt = NS * CH_PER_TILE + sid

    def _tail_fetch_idx():
        return pltpu.make_async_copy(
            dst_hbm.at[0, pl.ds(gt * CHUNK, CHUNK)], tidx_v.at[0], tsem
        )

    def _tail_fetch_rows():
        return pltpu.make_async_copy(
            w_hbm.at[pl.ds(gt * CHUNK, CHUNK), pl.ds(fbase, FHALF)],
            trows_v,
            tsem,
        )

    def _tail_scatter():
        return pltpu.make_async_copy(trows_v, acc_sh.at[tidx_v.at[0]], tsem)

    @pl.when(sid < TAIL_CH)
    def _tail_prefetch():
        _tail_fetch_idx().start()
        _tail_fetch_rows().start()

    zvec = jnp.zeros((16,), jnp.float32)

    def _zrow(r, carry):
        for c in range(FHALF // 16):
            zbuf[r, pl.ds(c * 16, 16)] = zvec
        return carry

    lax.fori_loop(0, CHUNK, _zrow, 0)

    def _zcopy(k):
        return pltpu.make_async_copy(
            zbuf, acc_sh.at[pl.ds(rbase + k * CHUNK, CHUNK), :], zsem
        )

    _zn = ROWS_PER_TILE // CHUNK
    _zn_last = ROWS_LAST // CHUNK

    @pl.when(sid < NS - 1)
    def _zero_main():
        for k in range(_zn):
            _zcopy(k).start()
        for k in range(_zn):
            _zcopy(k).wait()

    @pl.when(sid == NS - 1)
    def _zero_last():
        for k in range(_zn_last):
            _zcopy(k).start()
        pltpu.make_async_copy(
            zbuf.at[pl.ds(0, ROWS_LAST % CHUNK)],
            acc_sh.at[pl.ds(rbase + _zn_last * CHUNK, ROWS_LAST % CHUNK), :],
            zsem,
        ).start()
        for k in range(_zn_last):
            _zcopy(k).wait()
        pltpu.make_async_copy(
            zbuf.at[pl.ds(0, ROWS_LAST % CHUNK)],
            acc_sh.at[pl.ds(rbase + _zn_last * CHUNK, ROWS_LAST % CHUNK), :],
            zsem,
        ).wait()

    plsc.subcore_barrier()

    def round_body(t, carry):
        t0 = t * NBUF
        for b in range(NBUF):
            j = t0 + b
            _wait_fetch(cbase + j, b)
            _scatter(b).start(add=True)
        for b in range(NBUF):
            j = t0 + b

            @pl.when(j + NBUF < CH_PER_TILE)
            def _refill(j=j, b=b):
                _scatter(b).wait()
                _start_fetch(cbase + j + NBUF, b)

        return carry

    lax.fori_loop(0, NROUND, round_body, 0)

    @pl.when(sid < TAIL_CH)
    def _tail():
        _tail_fetch_idx().wait()
        _tail_fetch_rows().wait()
        _tail_scatter().start(add=True)

    for b in range(NBUF):
        _scatter(b).wait()

    @pl.when(sid < TAIL_CH)
    def _tail_drain():
        _tail_scatter().wait()

    plsc.subcore_barrier()

    @pl.when(sid < NS - 1)
    def _flush_main():
        pltpu.sync_copy(
            acc_sh.at[pl.ds(rbase, ROWS_PER_TILE), :],
            out_hbm.at[pl.ds(rbase, ROWS_PER_TILE), pl.ds(fbase, FHALF)],
        )

    @pl.when(sid == NS - 1)
    def _flush_last():
        pltpu.sync_copy(
            acc_sh.at[pl.ds(15 * ROWS_PER_TILE, ROWS_LAST), :],
            out_hbm.at[pl.ds(15 * ROWS_PER_TILE, ROWS_LAST),
                       pl.ds(fbase, FHALF)],
        )


def kernel(edge, edge_w, N, E, out_features):
    if edge.dtype != jnp.int32:
        edge = edge.astype(jnp.int32)
    return _sc_scatter(edge, edge_w)
